# baseline (speedup 1.0000x reference)
import functools

import jax
import jax.numpy as jnp
from jax import lax
from jax.experimental import pallas as pl
from jax.experimental.pallas import tpu as pltpu
from jax.experimental.pallas import tpu_sc as plsc

B = 16384
CONT_F = 13
CATE_F = 26
COMB_F = 325

NC = 2
NS = 16
NW = NC * NS
ROWS_PER_W = B // NW
CHUNK = 128
NCHUNK = ROWS_PER_W // CHUNK

COMB_N = COMB_F * CHUNK
CATE_N = CATE_F * CHUNK
CONT_N = CONT_F * CHUNK
PAD = 16

_mesh = plsc.VectorSubcoreMesh(core_axis_name="c", subcore_axis_name="s")


@functools.partial(
    pl.kernel,
    mesh=_mesh,
    out_type=jax.ShapeDtypeStruct((B,), jnp.float32),
    scratch_types=[
        pltpu.VMEM((COMB_N + PAD,), jnp.int32),
        pltpu.VMEM((COMB_N + PAD,), jnp.float32),
        pltpu.VMEM((CATE_N + PAD,), jnp.int32),
        pltpu.VMEM((CATE_N + PAD,), jnp.float32),
        pltpu.VMEM((CONT_N + PAD,), jnp.float32),
        pltpu.VMEM((16,), jnp.float32),
        pltpu.VMEM((48,), jnp.float32),
        pltpu.VMEM((256,), jnp.float32),
        pltpu.VMEM((32,), jnp.float32),
        pltpu.VMEM((ROWS_PER_W,), jnp.float32),
        pltpu.SemaphoreType.DMA,
    ],
)
def _poly2_sc(conts_f, cates_f, combs_f, cate_tab, comb_tab, masks_hbm,
              eye_hbm, out_hbm,
              comb_idx_v, comb_val_v, cate_idx_v, cate_val_v, cont_v,
              w_v, masks_v, eye_v, buf_v, out_v, sem):
    cid = lax.axis_index("c")
    sid = lax.axis_index("s")
    wid = sid * NC + cid
    base = wid * ROWS_PER_W

    comb_idx_v[pl.ds(COMB_N, PAD)] = jnp.zeros((PAD,), jnp.int32)
    cate_idx_v[pl.ds(CATE_N, PAD)] = jnp.zeros((PAD,), jnp.int32)
    cont_v[pl.ds(CONT_N, PAD)] = jnp.zeros((PAD,), jnp.float32)

    pltpu.sync_copy(masks_hbm, masks_v)
    pltpu.sync_copy(eye_hbm, eye_v)
    pltpu.sync_copy(cate_tab.at[pl.ds(0, 16)], w_v)
    m5 = masks_v[pl.ds(0, 16)]
    m10 = masks_v[pl.ds(16, 16)]
    w_vec = w_v[pl.ds(0, 16)] * masks_v[pl.ds(32, 16)]

    def chunk_body(c, carry):
        rb = base + c * CHUNK
        pltpu.sync_copy(combs_f.at[pl.ds(rb * COMB_F, COMB_N)],
                        comb_idx_v.at[pl.ds(0, COMB_N)])
        pltpu.sync_copy(cates_f.at[pl.ds(rb * CATE_F, CATE_N)],
                        cate_idx_v.at[pl.ds(0, CATE_N)])
        pltpu.sync_copy(conts_f.at[pl.ds(rb * CONT_F, CONT_N)],
                        cont_v.at[pl.ds(0, CONT_N)])

        cp_comb = pltpu.async_copy(comb_tab.at[comb_idx_v], comb_val_v, sem)
        cp_cate = pltpu.async_copy(cate_tab.at[cate_idx_v], cate_val_v, sem)
        cp_comb.wait()
        cp_cate.wait()

        def group_body(bs, carry2):
            def row_body(j, rowsum):
                r = bs * 16 + j
                co = r * COMB_F

                def comb_k_body(k, acc):
                    return acc + comb_val_v[pl.ds(co + k * 16, 16)]

                acc = lax.fori_loop(0, 20, comb_k_body,
                                    jnp.zeros((16,), jnp.float32))
                acc = acc + comb_val_v[pl.ds(co + 320, 16)] * m5
                acc = acc + cate_val_v[pl.ds(r * CATE_F, 16)]
                acc = acc + cate_val_v[pl.ds(r * CATE_F + 16, 16)] * m10
                acc = acc + cont_v[pl.ds(r * CONT_F, 16)] * w_vec
                s = acc
                for sh in (8, 4, 2, 1):
                    buf_v[pl.ds(0, 16)] = s
                    buf_v[pl.ds(16, 16)] = s
                    s = s + buf_v[pl.ds(sh, 16)]
                return rowsum + s * eye_v[pl.ds(j * 16, 16)]

            rowsum = lax.fori_loop(0, 16, row_body,
                                   jnp.zeros((16,), jnp.float32))
            out_v[pl.ds(c * CHUNK + bs * 16, 16)] = (
                1.0 / (1.0 + jnp.exp(-rowsum)))
            return carry2

        return lax.fori_loop(0, CHUNK // 16, group_body, carry)

    lax.fori_loop(0, NCHUNK, chunk_body, jnp.int32(0))

    pltpu.sync_copy(out_v, out_hbm.at[pl.ds(base, ROWS_PER_W)])


def kernel(conts, cates, combs, cate_table, comb_table):
    i16 = jnp.arange(16)
    masks = jnp.concatenate([(i16 < 5).astype(jnp.float32),
                             (i16 < 10).astype(jnp.float32),
                             (i16 < 13).astype(jnp.float32)])
    eye = jnp.eye(16, dtype=jnp.float32).reshape(-1)
    out = _poly2_sc(conts.reshape(-1),
                    cates.reshape(-1).astype(jnp.int32),
                    combs.reshape(-1).astype(jnp.int32),
                    cate_table.reshape(-1), comb_table.reshape(-1),
                    masks, eye)
    return out.reshape(B, 1)


# speedup vs baseline: 1.2667x; 1.2667x over previous
import functools

import jax
import jax.numpy as jnp
from jax import lax
from jax.experimental import pallas as pl
from jax.experimental.pallas import tpu as pltpu
from jax.experimental.pallas import tpu_sc as plsc

B = 16384
CONT_F = 13
CATE_F = 26
COMB_F = 325
H1 = 176

NC = 2
NS = 16
NW = NC * NS
ROWS_PER_W = B // NW
CHUNK = 128
NCHUNK = ROWS_PER_W // CHUNK
NG = CHUNK // 16

_mesh = plsc.VectorSubcoreMesh(core_axis_name="c", subcore_axis_name="s")


@functools.partial(
    pl.kernel,
    mesh=_mesh,
    out_type=jax.ShapeDtypeStruct((B,), jnp.float32),
    scratch_types=[
        pltpu.VMEM((COMB_F, CHUNK), jnp.int32),
        pltpu.VMEM((COMB_F, CHUNK), jnp.float32),
        pltpu.VMEM((CATE_F, CHUNK), jnp.int32),
        pltpu.VMEM((CATE_F, CHUNK), jnp.float32),
        pltpu.VMEM((2 * CONT_F, CHUNK), jnp.float32),
        pltpu.VMEM((CONT_F * 16,), jnp.float32),
        pltpu.VMEM((CHUNK,), jnp.float32),
        pltpu.VMEM((ROWS_PER_W,), jnp.float32),
        pltpu.SemaphoreType.DMA,
        pltpu.SemaphoreType.DMA,
        pltpu.SemaphoreType.DMA,
    ],
)
def _poly2_sc(conts_t, cates_t, combs_t, cate_tab, comb_tab, wbc_hbm,
              out_hbm,
              comb_idx_v, comb_val_v, cate_idx_v, cate_val_v, cont_v,
              w_v, part_v, out_v, sem_a, sem_b, sem_s):
    cid = lax.axis_index("c")
    sid = lax.axis_index("s")
    wid = sid * NC + cid
    base = wid * ROWS_PER_W

    pltpu.sync_copy(wbc_hbm, w_v)

    def stage(c, parity, sync):
        rb = base + c * CHUNK
        copies = [
            (combs_t.at[:, pl.ds(rb, CHUNK)], comb_idx_v),
            (cates_t.at[:, pl.ds(rb, CHUNK)], cate_idx_v),
            (conts_t.at[:, pl.ds(rb, CHUNK)],
             cont_v.at[pl.ds(parity * CONT_F, CONT_F), :]),
        ]
        if sync:
            for src, dst in copies:
                pltpu.sync_copy(src, dst)
        else:
            for src, dst in copies:
                pltpu.async_copy(src, dst, sem_s)

    def stage_drain():
        pltpu.make_async_copy(combs_t.at[:, pl.ds(0, CHUNK)],
                              comb_idx_v, sem_s).wait()
        pltpu.make_async_copy(cates_t.at[:, pl.ds(0, CHUNK)],
                              cate_idx_v, sem_s).wait()
        pltpu.make_async_copy(conts_t.at[:, pl.ds(0, CHUNK)],
                              cont_v.at[pl.ds(0, CONT_F), :], sem_s).wait()

    def fire_h1(sem):
        def body(f, carry):
            pltpu.async_copy(comb_tab.at[comb_idx_v.at[f]],
                             comb_val_v.at[f], sem)
            return carry
        lax.fori_loop(0, H1, body, jnp.int32(0))

    def fire_h2(sem):
        def body(f, carry):
            pltpu.async_copy(comb_tab.at[comb_idx_v.at[f]],
                             comb_val_v.at[f], sem)
            return carry
        lax.fori_loop(H1, COMB_F, body, jnp.int32(0))

        def body_c(f, carry):
            pltpu.async_copy(cate_tab.at[cate_idx_v.at[f]],
                             cate_val_v.at[f], sem)
            return carry
        lax.fori_loop(0, CATE_F, body_c, jnp.int32(0))

    def drain(n_comb_rows, n_cate_rows, sem):
        def body(f, carry):
            pltpu.make_async_copy(cate_tab.at[pl.ds(0, CHUNK)],
                                  comb_val_v.at[f], sem).wait()
            return carry
        lax.fori_loop(0, n_comb_rows + n_cate_rows, body, jnp.int32(0))

    def compute_h1():
        def group(bs, carry):
            so = bs * 16

            def body(f, acc):
                return acc + comb_val_v[f, pl.ds(so, 16)]

            acc = lax.fori_loop(0, H1, body, jnp.zeros((16,), jnp.float32))
            part_v[pl.ds(so, 16)] = acc
            return carry
        lax.fori_loop(0, NG, group, jnp.int32(0))

    def compute_h2(c, parity):
        def group(bs, carry):
            so = bs * 16

            def body(f, acc):
                return acc + comb_val_v[f, pl.ds(so, 16)]

            acc = lax.fori_loop(H1, COMB_F, body, part_v[pl.ds(so, 16)])

            def body_c(f, acc):
                return acc + cate_val_v[f, pl.ds(so, 16)]

            acc = lax.fori_loop(0, CATE_F, body_c, acc)

            def body_w(f, acc):
                return acc + (cont_v[parity * CONT_F + f, pl.ds(so, 16)]
                              * w_v[pl.ds(f * 16, 16)])

            acc = lax.fori_loop(0, CONT_F, body_w, acc)

            out_v[pl.ds(c * CHUNK + so, 16)] = 1.0 / (1.0 + jnp.exp(-acc))
            return carry
        lax.fori_loop(0, NG, group, jnp.int32(0))

    stage(0, 0, sync=True)
    fire_h1(sem_a)
    for c in range(NCHUNK):
        parity = c % 2
        fire_h2(sem_b)
        drain(H1, 0, sem_a)
        compute_h1()
        drain(COMB_F - H1, CATE_F, sem_b)
        if c < NCHUNK - 1:
            stage(c + 1, 1 - parity, sync=False)
            stage_drain()
            fire_h1(sem_a)
        compute_h2(c, parity)

    pltpu.sync_copy(out_v, out_hbm.at[pl.ds(base, ROWS_PER_W)])


def kernel(conts, cates, combs, cate_table, comb_table):
    wbc = jnp.repeat(cate_table[:CONT_F, 0], 16)
    out = _poly2_sc(conts.T, cates.T.astype(jnp.int32),
                    combs.T.astype(jnp.int32),
                    cate_table.reshape(-1), comb_table.reshape(-1), wbc)
    return out.reshape(B, 1)
